# baseline (speedup 1.0000x reference)
import functools

import jax
import jax.numpy as jnp
from jax import lax
from jax.experimental import pallas as pl
from jax.experimental.pallas import tpu as pltpu
from jax.experimental.pallas import tpu_sc as plsc

NUM_USERS = 1000000
NUM_ITEMS = 1000000
EMB = 64
BATCH = 16384

NC = 2
NS = 16
NW = NC * NS
LANES = 16
B_PER_W = BATCH // NW
IDX_CHUNK = 128
N_IDX_CHUNKS = B_PER_W // IDX_CHUNK


def _sc_kernel_body(uidx_hbm, iidx_hbm, uemb_hbm, iemb_hbm, ub_hbm, ib_hbm,
                    out_hbm,
                    uidx_v, iidx_v, urows_v, irows_v, ubv, ibv, out_v, sem):
    wid = lax.axis_index("s") * NC + lax.axis_index("c")
    base = pl.multiple_of(wid * B_PER_W, B_PER_W)

    pltpu.sync_copy(uidx_hbm.at[pl.ds(base, B_PER_W)], uidx_v)
    pltpu.sync_copy(iidx_hbm.at[pl.ds(base, B_PER_W)], iidx_v)

    copies = []
    for k in range(N_IDX_CHUNKS):
        s = pl.ds(k * IDX_CHUNK, IDX_CHUNK)
        copies.append(pltpu.async_copy(uemb_hbm.at[uidx_v.at[s]],
                                       urows_v.at[s], sem))
        copies.append(pltpu.async_copy(iemb_hbm.at[iidx_v.at[s]],
                                       irows_v.at[s], sem))
        copies.append(pltpu.async_copy(ub_hbm.at[uidx_v.at[s]],
                                       ubv.at[s], sem))
        copies.append(pltpu.async_copy(ib_hbm.at[iidx_v.at[s]],
                                       ibv.at[s], sem))
    for c in copies:
        c.wait()

    def chunk(j, _):
        start = pl.multiple_of(j * LANES, LANES)
        lanes = lax.iota(jnp.int32, LANES) + start
        acc = ubv[pl.ds(start, LANES)] + ibv[pl.ds(start, LANES)]
        for d in range(EMB):
            dvec = jnp.full((LANES,), d, jnp.int32)
            u = plsc.load_gather(urows_v, [lanes, dvec])
            v = plsc.load_gather(irows_v, [lanes, dvec])
            acc = acc + u * v
        out_v[pl.ds(start, LANES)] = acc
        return 0

    lax.fori_loop(0, B_PER_W // LANES, chunk, 0)

    pltpu.sync_copy(out_v, out_hbm.at[pl.ds(base, B_PER_W)])


@jax.jit
def _run(uidx, iidx, uemb, iemb, ub, ib):
    mesh = plsc.VectorSubcoreMesh(core_axis_name="c", subcore_axis_name="s")
    f = pl.kernel(
        _sc_kernel_body, mesh=mesh,
        out_type=jax.ShapeDtypeStruct((BATCH,), jnp.float32),
        scratch_types=[
            pltpu.VMEM((B_PER_W,), jnp.int32),
            pltpu.VMEM((B_PER_W,), jnp.int32),
            pltpu.VMEM((B_PER_W, EMB), jnp.float32),
            pltpu.VMEM((B_PER_W, EMB), jnp.float32),
            pltpu.VMEM((B_PER_W,), jnp.float32),
            pltpu.VMEM((B_PER_W,), jnp.float32),
            pltpu.VMEM((B_PER_W,), jnp.float32),
            pltpu.SemaphoreType.DMA,
        ],
        compiler_params=pltpu.CompilerParams(needs_layout_passes=False,
                                             use_tc_tiling_on_sc=False),
    )
    return f(uidx, iidx, uemb, iemb, ub, ib)


def kernel(user_idx, item_idx, user_embeddings, item_embeddings,
           user_biases, item_biases):
    uidx = user_idx.astype(jnp.int32)
    iidx = item_idx.astype(jnp.int32)
    ub = jnp.reshape(user_biases, (NUM_USERS,))
    ib = jnp.reshape(item_biases, (NUM_ITEMS,))
    return _run(uidx, iidx, user_embeddings, item_embeddings, ub, ib)
